# baseline (speedup 1.0000x reference)
import functools

import jax
import jax.numpy as jnp
from jax import lax
from jax.experimental import pallas as pl
from jax.experimental.pallas import tpu as pltpu
from jax.experimental.pallas import tpu_sc as plsc

_XLA_RELAYOUT = False
_CHUNK = 128
_SKEW = 133
_NBUF = 5
_DEPTH = 4


@functools.lru_cache(maxsize=None)
def _make_kernels(V, D, B, L):
    info = plsc.get_sparse_core_info()
    NC, NS = info.num_cores, info.num_subcores
    NW = NC * NS
    N = B * L
    mesh = plsc.VectorSubcoreMesh(core_axis_name="c", subcore_axis_name="s")

    n_full = V // _CHUNK
    rem = V - n_full * _CHUNK
    per_w = (n_full + NW - 1) // NW
    triples = (per_w + 2) // 3

    @functools.partial(
        pl.kernel,
        mesh=mesh,
        compiler_params=pltpu.CompilerParams(needs_layout_passes=False),
        out_type=jax.ShapeDtypeStruct((V * D,), jnp.float32),
        scratch_types=[pltpu.VMEM((D, _CHUNK), jnp.float32)] * 3
        + [pltpu.VMEM((_CHUNK * D,), jnp.float32)] * 3
        + [pltpu.SemaphoreType.DMA] * 6,
    )
    def k1(tt_hbm, tail_hbm, out_hbm, *scr1):
        wid = lax.axis_index("s") * NC + lax.axis_index("c")
        stages = scr1[:3]
        trs = scr1[3:6]
        isems = scr1[6:9]
        osems = scr1[9:12]
        lane = lax.broadcasted_iota(jnp.int32, (16,), 0)

        def fire_in(rb, b):
            pltpu.async_copy(
                tt_hbm.at[:, pl.ds(rb * _CHUNK, _CHUNK)], stages[b], isems[b]
            )

        def wait_in(b):
            pltpu.make_async_copy(
                tt_hbm.at[:, pl.ds(0, _CHUNK)], stages[b], isems[b]
            ).wait()

        def fire_out(rb, b):
            pltpu.async_copy(
                trs[b], out_hbm.at[pl.ds(rb * _CHUNK * D, _CHUNK * D)], osems[b]
            )

        def wait_out(b):
            pltpu.make_async_copy(
                trs[b], out_hbm.at[pl.ds(0, _CHUNK * D)], osems[b]
            ).wait()

        cvecs = [c0 + lane for c0 in range(0, D, 16)]

        def transpose_panel(b, n_rows):
            def ts(s, carry):
                rsh = (lane + s) & 15

                def tg(g, carry2):
                    rvec = g * 16 + rsh
                    rd = rvec * D
                    for cvec in cvecs:
                        val = plsc.load_gather(stages[b], [cvec, rvec])
                        plsc.store_scatter(trs[b], [rd + cvec], val)
                    return carry2

                lax.fori_loop(0, n_rows // 16, tg, 0)
                return carry

            lax.fori_loop(0, 16, ts, 0)

        @pl.when(wid < n_full)
        def _():
            fire_in(wid, 0)

        @pl.when(wid + NW < n_full)
        def _():
            fire_in(wid + NW, 1)

        def body(p, carry):
            for par in range(3):
                j = 3 * p + par
                rb = j * NW + wid

                @pl.when(rb < n_full)
                def _():
                    nrb = rb + 2 * NW

                    @pl.when(nrb < n_full)
                    def _():
                        fire_in(nrb, (par + 2) % 3)

                    wait_in(par)

                    @pl.when(j >= 3)
                    def _():
                        wait_out(par)

                    transpose_panel(par, _CHUNK)
                    fire_out(rb, par)

            return carry

        lax.fori_loop(0, triples, body, 0)
        wait_out(0)
        wait_out(1)
        wait_out(2)

        if rem:

            @pl.when(wid == 0)
            def _():
                pltpu.sync_copy(tail_hbm, stages[0])
                transpose_panel(0, _CHUNK)
                pltpu.sync_copy(
                    trs[0],
                    out_hbm.at[pl.ds((V - _CHUNK) * D, _CHUNK * D)],
                )

    DB = D // 8
    BB = B // _CHUNK
    n_chunks = N // (NW * _CHUNK)

    @functools.partial(
        pl.kernel,
        mesh=mesh,
        compiler_params=pltpu.CompilerParams(
            use_tc_tiling_on_sc=False, needs_layout_passes=False
        ),
        out_type=jax.ShapeDtypeStruct((L, DB, BB, 8 * _CHUNK), jnp.float32),
        scratch_types=[pltpu.VMEM((_CHUNK,), jnp.int32)] * _NBUF
        + [pltpu.VMEM((_CHUNK, D), jnp.float32)] * _NBUF
        + [pltpu.VMEM((DB, 8 * _CHUNK), jnp.float32)] * _NBUF
        + [pltpu.SemaphoreType.DMA] * (3 * _NBUF),
    )
    def k2(tokt_hbm, tbl_hbm, out_hbm, *scr):
        idxs = scr[:_NBUF]
        rows = scr[_NBUF : 2 * _NBUF]
        packs = scr[2 * _NBUF : 3 * _NBUF]
        sems = scr[3 * _NBUF :]
        isems = sems[:_NBUF]
        gsems = sems[_NBUF : 2 * _NBUF]
        osems = sems[2 * _NBUF :]
        wid = lax.axis_index("s") * NC + lax.axis_index("c")
        q0 = wid * n_chunks
        lane = lax.broadcasted_iota(jnp.int32, (16,), 0)
        dvecs = [d0 + lane for d0 in range(0, D, 16)]
        dv3s = [dv >> 3 for dv in dvecs]
        in2s = [(dv & 7) << 7 for dv in dvecs]

        def fire_idx(q, b):
            l = q // BB
            bb = q % BB
            pltpu.async_copy(
                tokt_hbm.at[pl.ds(l * B + bb * _CHUNK, _CHUNK)],
                idxs[b],
                isems[b],
            )

        def wait_idx(b):
            pltpu.make_async_copy(
                tokt_hbm.at[pl.ds(0, _CHUNK)], idxs[b], isems[b]
            ).wait()

        def fire_gather(b):
            pltpu.async_copy(tbl_hbm.at[idxs[b]], rows[b], gsems[b])

        def wait_gather(b):
            pltpu.make_async_copy(
                tbl_hbm.at[idxs[b]], rows[b], gsems[b]
            ).wait()

        def fire_out(q, b):
            l = q // BB
            bb = q % BB
            pltpu.async_copy(packs[b], out_hbm.at[l, :, bb], osems[b])

        def wait_out(b):
            pltpu.make_async_copy(
                packs[b], out_hbm.at[0, :, 0], osems[b]
            ).wait()

        for c in range(_NBUF):
            fire_idx(q0 + c, c)
        for c in range(_DEPTH):
            wait_idx(c)
            fire_gather(c)

        def body(p, carry):
            for b in range(_NBUF):
                j = p * _NBUF + b

                nb = (b + _DEPTH) % _NBUF

                @pl.when(j + _DEPTH < n_chunks)
                def _():
                    wait_idx(nb)
                    fire_gather(nb)

                wait_gather(b)

                @pl.when(j + _NBUF < n_chunks)
                def _():
                    fire_idx(q0 + j + _NBUF, b)

                @pl.when(j >= _NBUF)
                def _():
                    wait_out(b)

                def ts(s, carry2):
                    tsh = (lane + s) & 15

                    def tg(g, carry3):
                        tvec = g * 16 + tsh
                        for di in range(D // 16):
                            val = plsc.load_gather(rows[b], [tvec, dvecs[di]])
                            plsc.store_scatter(
                                packs[b], [dv3s[di], in2s[di] + tvec], val
                            )
                        return carry3

                    lax.fori_loop(0, _CHUNK // 16, tg, 0)
                    return carry2

                lax.fori_loop(0, 16, ts, 0)
                fire_out(q0 + j, b)
            return carry

        lax.fori_loop(0, n_chunks // _NBUF, body, 0)
        for b in range(_NBUF):
            wait_out(b)

    def run(tokens, table):
        if _XLA_RELAYOUT:
            tbl_flat = jnp.reshape(table, (V * D,))
        else:
            tt = table.T
            tail = lax.slice(table, (V - _CHUNK, 0), (V, D)).T
            tbl_flat = k1(tt, tail)
        tbl_lin = tbl_flat.reshape(V, D)
        tokt = tokens.T.reshape(N).astype(jnp.int32)
        out4 = k2(tokt, tbl_lin)
        out = (
            out4.reshape(L, DB, BB, 8, _CHUNK)
            .transpose(2, 4, 0, 1, 3)
            .reshape(B, L, D)
        )
        return out

    return run


def kernel(tokens, table):
    B, L = tokens.shape
    V, D = table.shape
    return _make_kernels(V, D, B, L)(tokens, table)


# speedup vs baseline: 1.2277x; 1.2277x over previous
import functools

import jax
import jax.numpy as jnp
from jax import lax
from jax.experimental import pallas as pl
from jax.experimental.pallas import tpu as pltpu
from jax.experimental.pallas import tpu_sc as plsc

_XLA_RELAYOUT = True
_CHUNK = 128
_SKEW = 133
_NBUF = 5
_DEPTH = 3


@functools.lru_cache(maxsize=None)
def _make_kernels(V, D, B, L):
    info = plsc.get_sparse_core_info()
    NC, NS = info.num_cores, info.num_subcores
    NW = NC * NS
    N = B * L
    mesh = plsc.VectorSubcoreMesh(core_axis_name="c", subcore_axis_name="s")

    n_full = V // _CHUNK
    rem = V - n_full * _CHUNK
    per_w = (n_full + NW - 1) // NW
    triples = (per_w + 2) // 3

    @functools.partial(
        pl.kernel,
        mesh=mesh,
        compiler_params=pltpu.CompilerParams(needs_layout_passes=False),
        out_type=jax.ShapeDtypeStruct((V * D,), jnp.float32),
        scratch_types=[pltpu.VMEM((D, _CHUNK), jnp.float32)] * 3
        + [pltpu.VMEM((_CHUNK * D,), jnp.float32)] * 3
        + [pltpu.SemaphoreType.DMA] * 6,
    )
    def k1(tt_hbm, tail_hbm, out_hbm, *scr1):
        wid = lax.axis_index("s") * NC + lax.axis_index("c")
        stages = scr1[:3]
        trs = scr1[3:6]
        isems = scr1[6:9]
        osems = scr1[9:12]
        lane = lax.broadcasted_iota(jnp.int32, (16,), 0)

        def fire_in(rb, b):
            pltpu.async_copy(
                tt_hbm.at[:, pl.ds(rb * _CHUNK, _CHUNK)], stages[b], isems[b]
            )

        def wait_in(b):
            pltpu.make_async_copy(
                tt_hbm.at[:, pl.ds(0, _CHUNK)], stages[b], isems[b]
            ).wait()

        def fire_out(rb, b):
            pltpu.async_copy(
                trs[b], out_hbm.at[pl.ds(rb * _CHUNK * D, _CHUNK * D)], osems[b]
            )

        def wait_out(b):
            pltpu.make_async_copy(
                trs[b], out_hbm.at[pl.ds(0, _CHUNK * D)], osems[b]
            ).wait()

        cvecs = [c0 + lane for c0 in range(0, D, 16)]

        def transpose_panel(b, n_rows):
            def ts(s, carry):
                rsh = (lane + s) & 15

                def tg(g, carry2):
                    rvec = g * 16 + rsh
                    rd = rvec * D
                    for cvec in cvecs:
                        val = plsc.load_gather(stages[b], [cvec, rvec])
                        plsc.store_scatter(trs[b], [rd + cvec], val)
                    return carry2

                lax.fori_loop(0, n_rows // 16, tg, 0)
                return carry

            lax.fori_loop(0, 16, ts, 0)

        @pl.when(wid < n_full)
        def _():
            fire_in(wid, 0)

        @pl.when(wid + NW < n_full)
        def _():
            fire_in(wid + NW, 1)

        def body(p, carry):
            for par in range(3):
                j = 3 * p + par
                rb = j * NW + wid

                @pl.when(rb < n_full)
                def _():
                    nrb = rb + 2 * NW

                    @pl.when(nrb < n_full)
                    def _():
                        fire_in(nrb, (par + 2) % 3)

                    wait_in(par)

                    @pl.when(j >= 3)
                    def _():
                        wait_out(par)

                    transpose_panel(par, _CHUNK)
                    fire_out(rb, par)

            return carry

        lax.fori_loop(0, triples, body, 0)
        wait_out(0)
        wait_out(1)
        wait_out(2)

        if rem:

            @pl.when(wid == 0)
            def _():
                pltpu.sync_copy(tail_hbm, stages[0])
                transpose_panel(0, _CHUNK)
                pltpu.sync_copy(
                    trs[0],
                    out_hbm.at[pl.ds((V - _CHUNK) * D, _CHUNK * D)],
                )

    DB = D // 8
    BB = B // _CHUNK
    n_chunks = N // (NW * _CHUNK)

    @functools.partial(
        pl.kernel,
        mesh=mesh,
        compiler_params=pltpu.CompilerParams(
            use_tc_tiling_on_sc=False, needs_layout_passes=False
        ),
        out_type=jax.ShapeDtypeStruct((L, DB, BB, 8 * _CHUNK), jnp.float32),
        scratch_types=[pltpu.VMEM((_CHUNK,), jnp.int32)] * _NBUF
        + [pltpu.VMEM((_CHUNK, D), jnp.float32)] * _NBUF
        + [pltpu.VMEM((DB, 8 * _CHUNK), jnp.float32)] * _NBUF
        + [pltpu.SemaphoreType.DMA] * (3 * _NBUF),
    )
    def k2(tokt_hbm, tbl_hbm, out_hbm, *scr):
        idxs = scr[:_NBUF]
        rows = scr[_NBUF : 2 * _NBUF]
        packs = scr[2 * _NBUF : 3 * _NBUF]
        sems = scr[3 * _NBUF :]
        isems = sems[:_NBUF]
        gsems = sems[_NBUF : 2 * _NBUF]
        osems = sems[2 * _NBUF :]
        wid = lax.axis_index("s") * NC + lax.axis_index("c")
        q0 = wid * n_chunks
        lane = lax.broadcasted_iota(jnp.int32, (16,), 0)
        dvecs = [d0 + lane for d0 in range(0, D, 16)]
        dv3s = [dv >> 3 for dv in dvecs]
        in2s = [(dv & 7) << 7 for dv in dvecs]

        def fire_idx(q, b):
            l = q // BB
            bb = q % BB
            pltpu.async_copy(
                tokt_hbm.at[pl.ds(l * B + bb * _CHUNK, _CHUNK)],
                idxs[b],
                isems[b],
            )

        def wait_idx(b):
            pltpu.make_async_copy(
                tokt_hbm.at[pl.ds(0, _CHUNK)], idxs[b], isems[b]
            ).wait()

        def fire_gather(b):
            pltpu.async_copy(tbl_hbm.at[idxs[b]], rows[b], gsems[b])

        def wait_gather(b):
            pltpu.make_async_copy(
                tbl_hbm.at[idxs[b]], rows[b], gsems[b]
            ).wait()

        def fire_out(q, b):
            l = q // BB
            bb = q % BB
            pltpu.async_copy(packs[b], out_hbm.at[l, :, bb], osems[b])

        def wait_out(b):
            pltpu.make_async_copy(
                packs[b], out_hbm.at[0, :, 0], osems[b]
            ).wait()

        for c in range(_NBUF):
            fire_idx(q0 + c, c)
        for c in range(_DEPTH):
            wait_idx(c)
            fire_gather(c)

        def body(p, carry):
            for b in range(_NBUF):
                j = p * _NBUF + b

                nb = (b + _DEPTH) % _NBUF

                @pl.when(j + _DEPTH < n_chunks)
                def _():
                    wait_idx(nb)
                    fire_gather(nb)

                wait_gather(b)

                @pl.when(j + _NBUF < n_chunks)
                def _():
                    fire_idx(q0 + j + _NBUF, b)

                @pl.when(j >= _NBUF)
                def _():
                    wait_out(b)

                def ts(s, carry2):
                    tsh = (lane + s) & 15

                    def tg(g, carry3):
                        tvec = g * 16 + tsh
                        for di in range(D // 16):
                            val = plsc.load_gather(rows[b], [tvec, dvecs[di]])
                            plsc.store_scatter(
                                packs[b], [dv3s[di], in2s[di] + tvec], val
                            )
                        return carry3

                    lax.fori_loop(0, _CHUNK // 16, tg, 0)
                    return carry2

                lax.fori_loop(0, 16, ts, 0)
                fire_out(q0 + j, b)
            return carry

        lax.fori_loop(0, n_chunks // _NBUF, body, 0)
        for b in range(_NBUF):
            wait_out(b)

    def run(tokens, table):
        if _XLA_RELAYOUT:
            from jax.experimental.layout import Format, Layout, with_layout_constraint

            tbl_lin = with_layout_constraint(
                table, Layout(major_to_minor=(0, 1), tiling=((8,),))
            )
        else:
            tt = table.T
            tail = lax.slice(table, (V - _CHUNK, 0), (V, D)).T
            tbl_flat = k1(tt, tail)
            tbl_lin = tbl_flat.reshape(V, D)
        tokt = tokens.T.reshape(N).astype(jnp.int32)
        out4 = k2(tokt, tbl_lin)
        out = (
            out4.reshape(L, DB, BB, 8, _CHUNK)
            .transpose(2, 4, 0, 1, 3)
            .reshape(B, L, D)
        )
        return out

    return run


def kernel(tokens, table):
    B, L = tokens.shape
    V, D = table.shape
    return _make_kernels(V, D, B, L)(tokens, table)
